# Initial kernel scaffold; baseline (speedup 1.0000x reference)
import jax
import jax.numpy as jnp
from jax.experimental import pallas as pl


def kernel(x):
    raise NotImplementedError("write your pallas kernel here")


# speedup vs baseline: 194.8802x; 194.8802x over previous
import functools

import jax
import jax.numpy as jnp
from jax import lax
from jax.experimental import pallas as pl
from jax.experimental.pallas import tpu as pltpu
from jax.experimental.pallas import tpu_sc as plsc

_B, _S, _D = 2, 8192, 2048
_R1 = 256
_G1 = _S // _R1
_R2 = 256
_G2 = _S // _R2


def _mag_min_body(x_ref, pmin_ref):
    m = (jnp.abs(x_ref[0]) + jnp.abs(x_ref[1])) * 0.5
    pmin_ref[pl.program_id(0)] = jnp.min(m)


_pass1 = pl.pallas_call(
    _mag_min_body,
    grid=(_G1,),
    in_specs=[pl.BlockSpec((_B, _R1, _D), lambda i: (0, i, 0))],
    out_specs=pl.BlockSpec(memory_space=pltpu.SMEM),
    out_shape=jax.ShapeDtypeStruct((_G1,), jnp.float32),
)


_sc_mesh = plsc.VectorSubcoreMesh(core_axis_name="c", subcore_axis_name="s")


@functools.partial(
    pl.kernel,
    mesh=_sc_mesh,
    out_type=jax.ShapeDtypeStruct((16,), jnp.float32),
    scratch_types=[
        pltpu.VMEM((_G1,), jnp.float32),
        pltpu.VMEM((16,), jnp.float32),
    ],
)
def _sc_merge(pmin_hbm, thr_hbm, vin, vout):
    c = lax.axis_index("c")
    s = lax.axis_index("s")

    @pl.when(jnp.logical_and(c == 0, s == 0))
    def _():
        pltpu.sync_copy(pmin_hbm, vin)
        v = vin[pl.ds(0, 16)]
        for k in range(1, _G1 // 16):
            v = jnp.minimum(v, vin[pl.ds(16 * k, 16)])
        t = v[0]
        for k in range(1, 16):
            t = jnp.minimum(t, v[k])
        vout[...] = jnp.full((16,), t, dtype=jnp.float32)
        pltpu.sync_copy(vout, thr_hbm)


def _mask_mul_body(thr_ref, x_ref, o_ref):
    t = thr_ref[0]
    m = (jnp.abs(x_ref[0]) + jnp.abs(x_ref[1])) * 0.5
    keep = (m >= t).astype(jnp.float32)
    o_ref[0] = x_ref[0] * keep
    o_ref[1] = x_ref[1] * keep


_pass2 = pl.pallas_call(
    _mask_mul_body,
    grid=(_G2,),
    in_specs=[
        pl.BlockSpec(memory_space=pltpu.SMEM),
        pl.BlockSpec((_B, _R2, _D), lambda i: (0, i, 0)),
    ],
    out_specs=pl.BlockSpec((_B, _R2, _D), lambda i: (0, i, 0)),
    out_shape=jax.ShapeDtypeStruct((_B, _S, _D), jnp.float32),
)


def kernel(x):
    partials = _pass1(x)
    thr = _sc_merge(partials)
    return _pass2(thr, x)
